# baseline (speedup 1.0000x reference)
import functools

import jax
import jax.numpy as jnp
from jax import lax
from jax.experimental import pallas as pl
from jax.experimental.pallas import tpu as pltpu
from jax.experimental.pallas import tpu_sc as plsc

NUM_CORES = 2
NUM_SUBCORES = 16
NUM_WORKERS = NUM_CORES * NUM_SUBCORES
BATCHES_PER_CHUNK = 2
NBUF = 4


@functools.lru_cache(maxsize=None)
def _make_gather(n_batch: int, seq: int, dim: int):
    rows_per_chunk = BATCHES_PER_CHUNK * seq
    assert n_batch % (NUM_WORKERS * BATCHES_PER_CHUNK) == 0
    batches_per_w = n_batch // NUM_WORKERS
    n_chunks = batches_per_w // BATCHES_PER_CHUNK
    assert n_chunks % NBUF == 0
    n_groups = n_chunks // NBUF

    mesh = plsc.VectorSubcoreMesh(
        core_axis_name="c", subcore_axis_name="s",
        num_cores=NUM_CORES, num_subcores=NUM_SUBCORES)

    @functools.partial(
        pl.kernel,
        out_type=jax.ShapeDtypeStruct((n_batch, seq, dim), jnp.float32),
        mesh=mesh,
        compiler_params=pltpu.CompilerParams(use_tc_tiling_on_sc=True),
        scratch_types=[
            pltpu.VMEM((n_chunks, rows_per_chunk), jnp.int32),
            pltpu.VMEM((NBUF, rows_per_chunk, dim), jnp.float32),
            pltpu.SemaphoreType.DMA,
            pltpu.SemaphoreType.DMA((NBUF,)),
            pltpu.SemaphoreType.DMA((NBUF,)),
        ],
    )
    def gather_kernel(idx_hbm, table_hbm, out_hbm, idx_v, buf, isem,
                      gsem, wsem):
        wid = lax.axis_index("s") * NUM_CORES + lax.axis_index("c")
        base = wid * batches_per_w

        pltpu.async_copy(idx_hbm.at[wid], idx_v, isem).wait()

        def gstart(b, c):
            pltpu.async_copy(table_hbm.at[idx_v.at[c]], buf.at[b],
                             gsem.at[b])

        def gwait(b):
            pltpu.make_async_copy(
                table_hbm.at[idx_v.at[0]], buf.at[b], gsem.at[b]).wait()

        def wstart(b, c):
            for r in range(BATCHES_PER_CHUNK):
                pltpu.async_copy(
                    buf.at[b, pl.ds(r * seq, seq)],
                    out_hbm.at[base + c * BATCHES_PER_CHUNK + r],
                    wsem.at[b])

        def wwait(b):
            for _ in range(BATCHES_PER_CHUNK):
                pltpu.make_async_copy(
                    buf.at[b, pl.ds(0, seq)], out_hbm.at[0],
                    wsem.at[b]).wait()

        for b in range(NBUF):
            gstart(b, b)

        @pl.loop(0, n_groups)
        def _group(g):
            c0 = g * NBUF
            for b in range(NBUF):
                gwait(b)
                wstart(b, c0 + b)

            @pl.when(g < n_groups - 1)
            def _next():
                for b in range(NBUF):
                    wwait(b)
                    gstart(b, c0 + NBUF + b)

        for b in range(NBUF):
            wwait(b)

    return gather_kernel


NUM_PARTS = 4


def kernel(indices, weight):
    n_batch, seq = indices.shape
    dim = weight.shape[-1]
    rows_per_chunk = BATCHES_PER_CHUNK * seq
    part_batches = n_batch // NUM_PARTS
    gather = _make_gather(part_batches, seq, dim)
    parts = []
    for p in range(NUM_PARTS):
        idx_p = indices[p * part_batches:(p + 1) * part_batches].reshape(
            NUM_WORKERS,
            part_batches * seq // (NUM_WORKERS * rows_per_chunk),
            rows_per_chunk)
        parts.append(gather(idx_p, weight))
    return jnp.concatenate(parts, axis=0)


# speedup vs baseline: 1.0366x; 1.0366x over previous
import functools

import jax
import jax.numpy as jnp
from jax import lax
from jax.experimental import pallas as pl
from jax.experimental.pallas import tpu as pltpu
from jax.experimental.pallas import tpu_sc as plsc

NUM_CORES = 2
NUM_SUBCORES = 16
NUM_WORKERS = NUM_CORES * NUM_SUBCORES
BATCHES_PER_CHUNK = 2
NBUF = 4


@functools.lru_cache(maxsize=None)
def _make_gather(n_batch: int, seq: int, dim: int):
    rows_per_chunk = BATCHES_PER_CHUNK * seq
    assert n_batch % (NUM_WORKERS * BATCHES_PER_CHUNK) == 0
    batches_per_w = n_batch // NUM_WORKERS
    n_chunks = batches_per_w // BATCHES_PER_CHUNK
    assert n_chunks % NBUF == 0
    n_groups = n_chunks // NBUF

    mesh = plsc.VectorSubcoreMesh(
        core_axis_name="c", subcore_axis_name="s",
        num_cores=NUM_CORES, num_subcores=NUM_SUBCORES)

    @functools.partial(
        pl.kernel,
        out_type=jax.ShapeDtypeStruct((n_batch, seq, dim), jnp.float32),
        mesh=mesh,
        compiler_params=pltpu.CompilerParams(use_tc_tiling_on_sc=True),
        scratch_types=[
            pltpu.VMEM((n_chunks, rows_per_chunk), jnp.int32),
            pltpu.VMEM((NBUF, rows_per_chunk, dim), jnp.float32),
            pltpu.SemaphoreType.DMA,
            pltpu.SemaphoreType.DMA((NBUF,)),
            pltpu.SemaphoreType.DMA((NBUF,)),
        ],
    )
    def gather_kernel(idx_hbm, table_hbm, out_hbm, idx_v, buf, isem,
                      gsem, wsem):
        wid = lax.axis_index("s") * NUM_CORES + lax.axis_index("c")
        base = wid * batches_per_w

        pltpu.async_copy(idx_hbm.at[wid], idx_v, isem).wait()

        def gstart(b, c):
            pltpu.async_copy(table_hbm.at[idx_v.at[c]], buf.at[b],
                             gsem.at[b])

        def gwait(b):
            pltpu.make_async_copy(
                table_hbm.at[idx_v.at[0]], buf.at[b], gsem.at[b]).wait()

        def wstart(b, c):
            for r in range(BATCHES_PER_CHUNK):
                pltpu.async_copy(
                    buf.at[b, pl.ds(r * seq, seq)],
                    out_hbm.at[base + c * BATCHES_PER_CHUNK + r],
                    wsem.at[b])

        def wwait(b):
            for _ in range(BATCHES_PER_CHUNK):
                pltpu.make_async_copy(
                    buf.at[b, pl.ds(0, seq)], out_hbm.at[0],
                    wsem.at[b]).wait()

        for b in range(NBUF):
            gstart(b, b)

        @pl.loop(0, n_groups)
        def _group(g):
            c0 = g * NBUF
            for b in range(NBUF):
                gwait(b)
                wstart(b, c0 + b)

            @pl.when(g < n_groups - 1)
            def _next():
                for b in range(NBUF):
                    wwait(b)
                    gstart(b, c0 + NBUF + b)

        for b in range(NBUF):
            wwait(b)

    return gather_kernel


NUM_PARTS = 4


def kernel(indices, weight):
    n_batch, seq = indices.shape
    dim = weight.shape[-1]
    rows_per_chunk = BATCHES_PER_CHUNK * seq
    part_batches = n_batch // NUM_PARTS
    gather = _make_gather(part_batches, seq, dim)
    parts = []
    for p in range(NUM_PARTS):
        idx_p = indices[p * part_batches:(p + 1) * part_batches].reshape(
            NUM_WORKERS,
            part_batches * seq // (NUM_WORKERS * rows_per_chunk),
            rows_per_chunk)
        parts.append(gather(idx_p, weight))
    out = lax.pad(parts[0], jnp.float32(0),
                  ((0, n_batch - part_batches, 0), (0, 0, 0), (0, 0, 0)))
    for p in range(1, NUM_PARTS):
        out = lax.dynamic_update_slice(out, parts[p],
                                       (p * part_batches, 0, 0))
    return out


# speedup vs baseline: 1.1293x; 1.0894x over previous
import functools

import jax
import jax.numpy as jnp
from jax import lax
from jax.experimental import pallas as pl
from jax.experimental.pallas import tpu as pltpu
from jax.experimental.pallas import tpu_sc as plsc

NUM_CORES = 2
NUM_SUBCORES = 16
NUM_WORKERS = NUM_CORES * NUM_SUBCORES
BATCHES_PER_CHUNK = 2
NBUF = 2

SC_BATCHES = 3712
TC_BLOCK = 8


@functools.lru_cache(maxsize=None)
def _make_sc_gather(n_batch: int, seq: int, dim: int):
    rows_per_chunk = BATCHES_PER_CHUNK * seq
    assert n_batch % (NUM_WORKERS * BATCHES_PER_CHUNK) == 0
    batches_per_w = n_batch // NUM_WORKERS
    n_chunks = batches_per_w // BATCHES_PER_CHUNK
    assert n_chunks % NBUF == 0
    n_groups = n_chunks // NBUF

    mesh = plsc.VectorSubcoreMesh(
        core_axis_name="c", subcore_axis_name="s",
        num_cores=NUM_CORES, num_subcores=NUM_SUBCORES)

    @functools.partial(
        pl.kernel,
        out_type=jax.ShapeDtypeStruct((n_batch, seq, dim), jnp.float32),
        mesh=mesh,
        scratch_types=[
            pltpu.VMEM((n_chunks, rows_per_chunk), jnp.int32),
            pltpu.VMEM((NBUF, rows_per_chunk, dim), jnp.float32),
            pltpu.SemaphoreType.DMA,
            pltpu.SemaphoreType.DMA((NBUF,)),
            pltpu.SemaphoreType.DMA((NBUF,)),
        ],
    )
    def sc_gather(idx_hbm, table_hbm, out_hbm, idx_v, buf, isem, gsem,
                  wsem):
        wid = lax.axis_index("s") * NUM_CORES + lax.axis_index("c")
        base = wid * batches_per_w

        pltpu.async_copy(idx_hbm.at[wid], idx_v, isem).wait()

        def gstart(b, c):
            pltpu.async_copy(table_hbm.at[idx_v.at[c]], buf.at[b],
                             gsem.at[b])

        def gwait(b):
            pltpu.make_async_copy(
                table_hbm.at[idx_v.at[0]], buf.at[b], gsem.at[b]).wait()

        def wstart(b, c):
            for r in range(BATCHES_PER_CHUNK):
                pltpu.async_copy(
                    buf.at[b, pl.ds(r * seq, seq)],
                    out_hbm.at[base + c * BATCHES_PER_CHUNK + r],
                    wsem.at[b])

        def wwait(b):
            for _ in range(BATCHES_PER_CHUNK):
                pltpu.make_async_copy(
                    buf.at[b, pl.ds(0, seq)], out_hbm.at[0],
                    wsem.at[b]).wait()

        for b in range(NBUF):
            gstart(b, b)

        @pl.loop(0, n_groups)
        def _group(g):
            c0 = g * NBUF
            for b in range(NBUF):
                gwait(b)
                wstart(b, c0 + b)

            @pl.when(g < n_groups - 1)
            def _next():
                for b in range(NBUF):
                    wwait(b)
                    gstart(b, c0 + NBUF + b)

        for b in range(NBUF):
            wwait(b)

    return sc_gather


@functools.lru_cache(maxsize=None)
def _make_tc_gather(n_batch: int, tc_batch0: int, seq: int, dim: int,
                    n_table: int):
    tc_batches = n_batch - tc_batch0
    assert tc_batches % TC_BLOCK == 0
    block0 = tc_batch0 // TC_BLOCK

    def body(idx_sref, table_ref, out_ref):
        for b in range(TC_BLOCK):
            def row(s, carry, b=b):
                r = idx_sref[0, 0, b * seq + s]
                out_ref[b, pl.ds(s, 1)] = table_ref[pl.ds(r, 1), :]
                return carry

            lax.fori_loop(0, seq, row, 0, unroll=10)

    return pl.pallas_call(
        body,
        grid=(tc_batches // TC_BLOCK,),
        in_specs=[
            pl.BlockSpec((1, 1, TC_BLOCK * seq), lambda i: (i, 0, 0),
                         memory_space=pltpu.SMEM),
            pl.BlockSpec((n_table, dim), lambda i: (0, 0)),
        ],
        out_specs=pl.BlockSpec((TC_BLOCK, seq, dim),
                               lambda i: (block0 + i, 0, 0)),
        out_shape=jax.ShapeDtypeStruct((n_batch, seq, dim), jnp.float32),
    )


def kernel(indices, weight):
    n_batch, seq = indices.shape
    dim = weight.shape[-1]
    rows_per_chunk = BATCHES_PER_CHUNK * seq
    sc_idx = indices[:SC_BATCHES].reshape(
        NUM_WORKERS, SC_BATCHES * seq // (NUM_WORKERS * rows_per_chunk),
        rows_per_chunk)
    sc_out = _make_sc_gather(SC_BATCHES, seq, dim)(sc_idx, weight)
    tc_full = _make_tc_gather(n_batch, SC_BATCHES, seq, dim,
                              weight.shape[0])(
        indices[SC_BATCHES:].reshape(-1, 1, TC_BLOCK * seq), weight)
    return lax.dynamic_update_slice(tc_full, sc_out, (0, 0, 0))


# speedup vs baseline: 1.8267x; 1.6175x over previous
import functools

import jax
import jax.numpy as jnp
from jax import lax
from jax.experimental import pallas as pl
from jax.experimental.pallas import tpu as pltpu
from jax.experimental.pallas import tpu_sc as plsc

NUM_CORES = 2
NUM_SUBCORES = 16
NUM_WORKERS = NUM_CORES * NUM_SUBCORES
BATCHES_PER_CHUNK = 2
NBUF = 4


@functools.lru_cache(maxsize=None)
def _make_gather(n_batch: int, seq: int, dim: int):
    rows_per_chunk = BATCHES_PER_CHUNK * seq
    assert n_batch % (NUM_WORKERS * BATCHES_PER_CHUNK) == 0
    batches_per_w = n_batch // NUM_WORKERS
    n_chunks = batches_per_w // BATCHES_PER_CHUNK
    assert n_chunks % NBUF == 0
    n_groups = n_chunks // NBUF

    mesh = plsc.VectorSubcoreMesh(
        core_axis_name="c", subcore_axis_name="s",
        num_cores=NUM_CORES, num_subcores=NUM_SUBCORES)

    @functools.partial(
        pl.kernel,
        out_type=jax.ShapeDtypeStruct((n_batch, seq, dim), jnp.float32),
        mesh=mesh,
        scratch_types=[
            pltpu.VMEM((n_chunks, rows_per_chunk), jnp.int32),
            pltpu.VMEM((NBUF, rows_per_chunk, dim), jnp.float32),
            pltpu.SemaphoreType.DMA,
            pltpu.SemaphoreType.DMA((NBUF,)),
            pltpu.SemaphoreType.DMA((NBUF,)),
        ],
    )
    def gather_kernel(idx_hbm, table_hbm, out_hbm, idx_v, buf, isem,
                      gsem, wsem):
        wid = lax.axis_index("s") * NUM_CORES + lax.axis_index("c")
        base = wid * batches_per_w

        pltpu.async_copy(idx_hbm.at[wid], idx_v, isem).wait()

        def gstart(b, c):
            pltpu.async_copy(table_hbm.at[idx_v.at[c]], buf.at[b],
                             gsem.at[b])

        def gwait(b):
            pltpu.make_async_copy(
                table_hbm.at[idx_v.at[0]], buf.at[b], gsem.at[b]).wait()

        def wstart(b, c):
            for r in range(BATCHES_PER_CHUNK):
                pltpu.async_copy(
                    buf.at[b, pl.ds(r * seq, seq)],
                    out_hbm.at[base + c * BATCHES_PER_CHUNK + r],
                    wsem.at[b])

        def wwait(b):
            for _ in range(BATCHES_PER_CHUNK):
                pltpu.make_async_copy(
                    buf.at[b, pl.ds(0, seq)], out_hbm.at[0],
                    wsem.at[b]).wait()

        for b in range(NBUF):
            gstart(b, b)

        @pl.loop(0, n_groups)
        def _group(g):
            c0 = g * NBUF
            for b in range(NBUF):
                gwait(b)
                wstart(b, c0 + b)

            @pl.when(g < n_groups - 1)
            def _next():
                for b in range(NBUF):
                    wwait(b)
                    gstart(b, c0 + NBUF + b)

        for b in range(NBUF):
            wwait(b)

    return gather_kernel


def kernel(indices, weight):
    n_batch, seq = indices.shape
    dim = weight.shape[-1]
    rows_per_chunk = BATCHES_PER_CHUNK * seq
    idx_grouped = indices.reshape(
        NUM_WORKERS, indices.size // (NUM_WORKERS * rows_per_chunk),
        rows_per_chunk)
    return _make_gather(n_batch, seq, dim)(idx_grouped, weight)


# speedup vs baseline: 1.8275x; 1.0004x over previous
import functools

import jax
import jax.numpy as jnp
from jax import lax
from jax.experimental import pallas as pl
from jax.experimental.pallas import tpu as pltpu
from jax.experimental.pallas import tpu_sc as plsc

NUM_CORES = 2
NUM_SUBCORES = 16
NUM_WORKERS = NUM_CORES * NUM_SUBCORES
BATCHES_PER_CHUNK = 2
NBUF = 8


@functools.lru_cache(maxsize=None)
def _make_gather(n_batch: int, seq: int, dim: int):
    rows_per_chunk = BATCHES_PER_CHUNK * seq
    assert n_batch % (NUM_WORKERS * BATCHES_PER_CHUNK) == 0
    batches_per_w = n_batch // NUM_WORKERS
    n_chunks = batches_per_w // BATCHES_PER_CHUNK
    assert n_chunks % NBUF == 0
    n_groups = n_chunks // NBUF

    mesh = plsc.VectorSubcoreMesh(
        core_axis_name="c", subcore_axis_name="s",
        num_cores=NUM_CORES, num_subcores=NUM_SUBCORES)

    @functools.partial(
        pl.kernel,
        out_type=jax.ShapeDtypeStruct((n_batch, seq, dim), jnp.float32),
        mesh=mesh,
        scratch_types=[
            pltpu.VMEM((n_chunks, rows_per_chunk), jnp.int32),
            pltpu.VMEM((NBUF, rows_per_chunk, dim), jnp.float32),
            pltpu.SemaphoreType.DMA,
            pltpu.SemaphoreType.DMA((NBUF,)),
            pltpu.SemaphoreType.DMA((NBUF,)),
        ],
    )
    def gather_kernel(idx_hbm, table_hbm, out_hbm, idx_v, buf, isem,
                      gsem, wsem):
        wid = lax.axis_index("s") * NUM_CORES + lax.axis_index("c")
        base = wid * batches_per_w

        pltpu.async_copy(idx_hbm.at[wid], idx_v, isem).wait()

        def gstart(b, c):
            pltpu.async_copy(table_hbm.at[idx_v.at[c]], buf.at[b],
                             gsem.at[b])

        def gwait(b):
            pltpu.make_async_copy(
                table_hbm.at[idx_v.at[0]], buf.at[b], gsem.at[b]).wait()

        def wstart(b, c):
            for r in range(BATCHES_PER_CHUNK):
                pltpu.async_copy(
                    buf.at[b, pl.ds(r * seq, seq)],
                    out_hbm.at[base + c * BATCHES_PER_CHUNK + r],
                    wsem.at[b])

        def wwait(b):
            for _ in range(BATCHES_PER_CHUNK):
                pltpu.make_async_copy(
                    buf.at[b, pl.ds(0, seq)], out_hbm.at[0],
                    wsem.at[b]).wait()

        for b in range(NBUF):
            gstart(b, b)

        @pl.loop(0, n_groups)
        def _group(g):
            c0 = g * NBUF
            for b in range(NBUF):
                gwait(b)
                wstart(b, c0 + b)

            @pl.when(g < n_groups - 1)
            def _next():
                for b in range(NBUF):
                    wwait(b)
                    gstart(b, c0 + NBUF + b)

        for b in range(NBUF):
            wwait(b)

    return gather_kernel


def kernel(indices, weight):
    n_batch, seq = indices.shape
    dim = weight.shape[-1]
    rows_per_chunk = BATCHES_PER_CHUNK * seq
    idx_grouped = indices.reshape(
        NUM_WORKERS, indices.size // (NUM_WORKERS * rows_per_chunk),
        rows_per_chunk)
    return _make_gather(n_batch, seq, dim)(idx_grouped, weight)
